# baseline (speedup 1.0000x reference)
import functools

import jax
import jax.numpy as jnp
from jax import lax
from jax.experimental import pallas as pl
from jax.experimental.pallas import tpu as pltpu
from jax.experimental.pallas import tpu_sc as plsc

_K = 26214
_ROWS = 128
_COLS = 32768
_CHUNKS = _COLS // 16
_ROWS_PER_SUBCORE = 4


def _ukey(b):
    sign = jnp.uint32(0x80000000)
    return jnp.where(b >= sign, ~b, b | sign)


def _sc_body(x_hbm, o_hbm, xbuf, hist):
    c = lax.axis_index("c")
    s = lax.axis_index("s")
    wid = s * 2 + c
    lanes = lax.iota(jnp.int32, 16)
    ones = jnp.ones((16,), jnp.int32)

    for j in range(_ROWS_PER_SUBCORE):
        row = wid * _ROWS_PER_SUBCORE + j
        pltpu.sync_copy(x_hbm.at[row], xbuf)

        prefix = jnp.uint32(0)
        rank = jnp.int32(_K)
        for shift in (24, 16, 8, 0):

            def _zero(b, carry):
                hist[pl.ds(b * 16, 16)] = jnp.zeros((16,), jnp.int32)
                return carry

            lax.fori_loop(0, 256, _zero, 0)

            def _hist(i, carry, _shift=shift, _prefix=prefix):
                u = _ukey(xbuf[pl.ds(i * 16, 16)])
                d = ((u >> jnp.uint32(_shift)) & jnp.uint32(0xFF)).astype(
                    jnp.int32
                )
                slot = d * jnp.int32(16) + lanes
                if _shift == 24:
                    plsc.addupdate_scatter(hist, [slot], ones)
                else:
                    hi = jnp.uint32(_shift + 8)
                    active = (u >> hi) == (_prefix >> hi)
                    plsc.addupdate_scatter(hist, [slot], ones, mask=active)
                return carry

            lax.fori_loop(0, _CHUNKS, _hist, 0)

            def _scan(i, carry, _rank=rank):
                cum, chosen, rnew = carry
                b = 255 - i
                h = jnp.sum(hist[pl.ds(b * 16, 16)])
                cum2 = cum + h
                found = (cum < _rank) & (cum2 >= _rank)
                chosen = jnp.where(found, b, chosen)
                rnew = jnp.where(found, _rank - cum, rnew)
                return (cum2, chosen, rnew)

            _, chosen, rank = lax.fori_loop(
                0, 256, _scan, (jnp.int32(0), jnp.int32(0), rank)
            )
            prefix = prefix | (chosen.astype(jnp.uint32) << jnp.uint32(shift))

        def _mask(i, carry, _prefix=prefix):
            sl = pl.ds(i * 16, 16)
            v = xbuf[sl]
            keep = _ukey(v) >= _prefix
            xbuf[sl] = jnp.where(keep, v, jnp.uint32(0))
            return carry

        lax.fori_loop(0, _CHUNKS, _mask, 0)
        pltpu.sync_copy(xbuf, o_hbm.at[row])


def kernel(x):
    f = pl.kernel(
        _sc_body,
        out_type=jax.ShapeDtypeStruct((_ROWS, _COLS), jnp.uint32),
        mesh=plsc.VectorSubcoreMesh(core_axis_name="c", subcore_axis_name="s"),
        compiler_params=pltpu.CompilerParams(needs_layout_passes=False),
        scratch_types=[
            pltpu.VMEM((_COLS,), jnp.uint32),
            pltpu.VMEM((4096,), jnp.int32),
        ],
    )
    xu = jax.lax.bitcast_convert_type(x, jnp.uint32)
    return jax.lax.bitcast_convert_type(f(xu), jnp.float32)


# speedup vs baseline: 3.3191x; 3.3191x over previous
import jax
import jax.numpy as jnp
from jax import lax
from jax.experimental import pallas as pl
from jax.experimental.pallas import tpu as pltpu
from jax.experimental.pallas import tpu_sc as plsc

_K = 26214
_ROWS = 128
_COLS = 32768
_CHUNKS = _COLS // 16
_ROWS_PER_SUBCORE = 4
_NHIST = 4
_HSTRIDE = 4096


def _ukey(b):
    sign = jnp.uint32(0x80000000)
    return jnp.where(b >= sign, ~b, b | sign)


def _sc_body(x_hbm, o_hbm, xbuf, hist):
    c = lax.axis_index("c")
    s = lax.axis_index("s")
    wid = s * 2 + c
    lanes = lax.iota(jnp.int32, 16)
    ones = jnp.ones((16,), jnp.int32)

    for j in range(_ROWS_PER_SUBCORE):
        row = wid * _ROWS_PER_SUBCORE + j
        pltpu.sync_copy(x_hbm.at[row], xbuf)

        prefix = jnp.uint32(0)
        rank = jnp.int32(_K)
        for shift in (24, 16, 8, 0):

            @plsc.parallel_loop(0, _NHIST * 256, unroll=8)
            def _zero(i):
                hist[pl.ds(i * 16, 16)] = jnp.zeros((16,), jnp.int32)

            def _hist(i, _shift=shift, _prefix=prefix):
                u = _ukey(xbuf[pl.ds(i * 16, 16)])
                d = ((u >> jnp.uint32(_shift)) & jnp.uint32(0xFF)).astype(
                    jnp.int32
                )
                slot = d * jnp.int32(16) + lanes + (i & 3) * jnp.int32(_HSTRIDE)
                if _shift == 24:
                    plsc.addupdate_scatter(hist, [slot], ones)
                else:
                    hi = jnp.uint32(_shift + 8)
                    active = (u >> hi) == (_prefix >> hi)
                    plsc.addupdate_scatter(hist, [slot], ones, mask=active)

            plsc.parallel_loop(0, _CHUNKS, unroll=4)(_hist)

            def _scan(i, carry, _rank=rank):
                cum, chosen, rnew = carry
                b = 255 - i
                base = b * 16
                hv = (
                    hist[pl.ds(base, 16)]
                    + hist[pl.ds(base + _HSTRIDE, 16)]
                    + hist[pl.ds(base + 2 * _HSTRIDE, 16)]
                    + hist[pl.ds(base + 3 * _HSTRIDE, 16)]
                )
                cum2 = cum + jnp.sum(hv)
                found = (cum < _rank) & (cum2 >= _rank)
                chosen = jnp.where(found, b, chosen)
                rnew = jnp.where(found, _rank - cum, rnew)
                return (cum2, chosen, rnew)

            _, chosen, rank = plsc.parallel_loop(
                0, 256, unroll=4, carry=(jnp.int32(0), jnp.int32(0), rank)
            )(_scan)
            prefix = prefix | (chosen.astype(jnp.uint32) << jnp.uint32(shift))

        @plsc.parallel_loop(0, _CHUNKS, unroll=8)
        def _mask(i, _prefix=prefix):
            sl = pl.ds(i * 16, 16)
            v = xbuf[sl]
            keep = _ukey(v) >= _prefix
            xbuf[sl] = jnp.where(keep, v, jnp.uint32(0))

        pltpu.sync_copy(xbuf, o_hbm.at[row])


def kernel(x):
    f = pl.kernel(
        _sc_body,
        out_type=jax.ShapeDtypeStruct((_ROWS, _COLS), jnp.uint32),
        mesh=plsc.VectorSubcoreMesh(core_axis_name="c", subcore_axis_name="s"),
        compiler_params=pltpu.CompilerParams(needs_layout_passes=False),
        scratch_types=[
            pltpu.VMEM((_COLS,), jnp.uint32),
            pltpu.VMEM((_NHIST * _HSTRIDE,), jnp.int32),
        ],
    )
    xu = jax.lax.bitcast_convert_type(x, jnp.uint32)
    return jax.lax.bitcast_convert_type(f(xu), jnp.float32)
